# Initial kernel scaffold; baseline (speedup 1.0000x reference)
import jax
import jax.numpy as jnp
from jax.experimental import pallas as pl


def kernel(x, W, b):
    raise NotImplementedError("write your pallas kernel here")


# speedup vs baseline: 1.6979x; 1.6979x over previous
import functools

import jax
import jax.numpy as jnp
from jax.experimental import pallas as pl
from jax.experimental.pallas import tpu as pltpu

N, C, H, WD = 2, 192, 224, 224
CO = 384
TILE_H = 8
WP = 240
HP = H + 2
M = TILE_H * WP


def _conv_body(x_ref, w_ref, b_ref, o_ref):
    full = x_ref[0].reshape((TILE_H + 2) * WP, C)
    zrow = jnp.zeros((1, C), jnp.bfloat16)
    shifted = (
        jnp.concatenate([zrow, full[:-1]], axis=0),
        full,
        jnp.concatenate([full[1:], zrow], axis=0),
    )
    acc = jnp.broadcast_to(b_ref[0][None, :], (M, CO)).astype(jnp.float32)
    for kh in range(3):
        for kw in range(3):
            lhs = shifted[kw][kh * WP:kh * WP + M]
            acc = acc + jnp.dot(lhs, w_ref[kh * 3 + kw],
                                preferred_element_type=jnp.float32)
    o_ref[0] = acc.reshape(TILE_H, WP, CO)[:, 8:8 + WD, :]


@jax.jit
def kernel(x, W, b):
    xt = jnp.transpose(x, (0, 2, 3, 1)).astype(jnp.bfloat16)
    xp = jnp.pad(xt, ((0, 0), (1, 1), (8, 8), (0, 0)))
    wt = jnp.transpose(W, (2, 3, 1, 0)).reshape(9, C, CO).astype(jnp.bfloat16)
    b2 = b.reshape(1, CO)

    n_tiles = H // TILE_H
    out_nhwc = pl.pallas_call(
        _conv_body,
        grid=(N, n_tiles),
        in_specs=[
            pl.BlockSpec(
                (pl.Element(1), pl.Element(TILE_H + 2),
                 pl.Element(WP), pl.Element(C)),
                lambda n, i: (n, i * TILE_H, 0, 0),
            ),
            pl.BlockSpec((9, C, CO), lambda n, i: (0, 0, 0)),
            pl.BlockSpec((1, CO), lambda n, i: (0, 0)),
        ],
        out_specs=pl.BlockSpec((1, TILE_H, WD, CO),
                               lambda n, i: (n, i, 0, 0)),
        out_shape=jax.ShapeDtypeStruct((N, H, WD, CO), jnp.float32),
        compiler_params=pltpu.CompilerParams(
            dimension_semantics=("parallel", "parallel"),
        ),
    )(xp, wt, b2)
    return jnp.transpose(out_nhwc, (0, 3, 1, 2))
